# baseline (speedup 1.0000x reference)
import functools

import jax
import jax.numpy as jnp
from jax import lax
from jax.experimental import pallas as pl
from jax.experimental.pallas import tpu as pltpu
from jax.experimental.pallas import tpu_sc as plsc

B = 4
N = 50000
D = 128
H = D // 2
NC, NS, L = 2, 16, 16
NW = NC * NS
ROWS_PER_W = 1568
NPAD = NW * ROWS_PER_W
TC_BLK = 1568
K = 56
G = 2 * K
NTILES = ROWS_PER_W // K
NTOT = B * N * D


def _bf16_bits(x):
    h = lax.bitcast_convert_type(x.astype(jnp.bfloat16), jnp.uint16)
    return h.astype(jnp.uint32)


def _dense_body(c1_ref, c2_ref, sq_ref, sp_ref):
    i = pl.program_id(0)
    d = c2_ref[...] - c1_ref[...]
    row = lax.broadcasted_iota(jnp.int32, (1, TC_BLK, 1), 1) + i * TC_BLK
    d = jnp.where(row < N, d, 0.0)
    s = jnp.sum(d, axis=0)
    packed = _bf16_bits(s[:, :H]) | (_bf16_bits(s[:, H:]) << 16)
    sp_ref[...] = lax.bitcast_convert_type(packed, jnp.int32)

    @pl.when(i == 0)
    def _():
        sq_ref[...] = jnp.zeros_like(sq_ref)

    sq_ref[...] += jnp.sum(d * d)


_dense_call = pl.pallas_call(
    _dense_body,
    grid=(NPAD // TC_BLK,),
    in_specs=[
        pl.BlockSpec((B, TC_BLK, D), lambda i: (0, i, 0)),
        pl.BlockSpec((B, TC_BLK, D), lambda i: (0, i, 0)),
    ],
    out_specs=[
        pl.BlockSpec((1, 1), lambda i: (0, 0)),
        pl.BlockSpec((TC_BLK, H), lambda i: (i, 0)),
    ],
    out_shape=[
        jax.ShapeDtypeStruct((1, 1), jnp.float32),
        jax.ShapeDtypeStruct((NPAD, H), jnp.int32),
    ],
)


def _dot_body(u_ref, s_ref, o_ref):
    i = pl.program_id(0)

    @pl.when(i == 0)
    def _():
        o_ref[...] = jnp.zeros_like(o_ref)

    uw = lax.bitcast_convert_type(u_ref[...], jnp.uint32)
    sw = lax.bitcast_convert_type(s_ref[...], jnp.uint32)
    ulo = lax.bitcast_convert_type(uw << 16, jnp.float32)
    uhi = lax.bitcast_convert_type(uw & jnp.uint32(0xFFFF0000), jnp.float32)
    slo = lax.bitcast_convert_type(sw << 16, jnp.float32)
    shi = lax.bitcast_convert_type(sw & jnp.uint32(0xFFFF0000), jnp.float32)
    o_ref[...] += jnp.sum(ulo * slo) + jnp.sum(uhi * shi)


_dot_call = pl.pallas_call(
    _dot_body,
    grid=(NPAD // TC_BLK,),
    in_specs=[
        pl.BlockSpec((TC_BLK, H), lambda i: (i, 0)),
        pl.BlockSpec((TC_BLK, H), lambda i: (i, 0)),
    ],
    out_specs=pl.BlockSpec((1, 1), lambda i: (0, 0)),
    out_shape=jax.ShapeDtypeStruct((1, 1), jnp.float32),
)


def _sc_u_body(c1f, c2f, idx_hbm, up_hbm, p_hbm,
               idx_v, rc1, rc2, ub, outv, gsems, wsems):
    wid = lax.axis_index("s") * NC + lax.axis_index("c")
    base = wid * ROWS_PER_W

    pltpu.sync_copy(idx_hbm.at[pl.ds(2 * base, 2 * ROWS_PER_W)], idx_v)

    def fire(t):
        buf = t % 2
        sl = pl.ds(t * G, G)
        return (
            pltpu.async_copy(c1f.at[idx_v.at[sl]], rc1.at[buf],
                             gsems.at[buf, 0]),
            pltpu.async_copy(c2f.at[idx_v.at[sl]], rc2.at[buf],
                             gsems.at[buf, 1]),
        )

    half = jnp.int32(0x8000)
    himask = jnp.int32(-65536)
    acc2 = jnp.zeros((L,), jnp.float32)
    handles = {0: fire(0)}
    uwrites = {}
    for t in range(NTILES):
        if t + 1 < NTILES:
            if (t - 1) in uwrites:
                uwrites.pop(t - 1).wait()
            handles[t + 1] = fire(t + 1)
        for h in handles.pop(t):
            h.wait()
        buf = t % 2

        def row_body(r, racc, buf=buf):
            r2 = racc
            u = []
            for c in range(D // L):
                sl = pl.ds(c * L, L)
                uc = (rc2[buf, 2 * r, sl] - rc1[buf, 2 * r, sl]) + \
                     (rc2[buf, 2 * r + 1, sl] - rc1[buf, 2 * r + 1, sl])
                u.append(uc)
                r2 = r2 + uc * uc
            for c in range(H // L):
                wlo = lax.shift_right_logical(
                    lax.bitcast_convert_type(u[c], jnp.int32) + half, 16)
                whi = (lax.bitcast_convert_type(u[c + H // L], jnp.int32)
                       + half) & himask
                ub[buf, r, pl.ds(c * L, L)] = wlo | whi
            return r2

        acc2 = lax.fori_loop(0, K, row_body, acc2)
        uwrites[t] = pltpu.async_copy(
            ub.at[buf], up_hbm.at[pl.ds(base + t * K, K)], wsems.at[buf])

    for t in sorted(uwrites):
        uwrites.pop(t).wait()
    outv[...] = acc2
    pltpu.sync_copy(outv, p_hbm.at[wid])


@functools.cache
def _sc_u_call():
    mesh = plsc.VectorSubcoreMesh(core_axis_name="c", subcore_axis_name="s")
    return pl.kernel(
        _sc_u_body,
        out_type=(
            jax.ShapeDtypeStruct((NPAD, H), jnp.int32),
            jax.ShapeDtypeStruct((NW, L), jnp.float32),
        ),
        mesh=mesh,
        scratch_types=[
            pltpu.VMEM((2 * ROWS_PER_W,), jnp.int32),
            pltpu.VMEM((2, G, D), jnp.float32),
            pltpu.VMEM((2, G, D), jnp.float32),
            pltpu.VMEM((2, K, H), jnp.int32),
            pltpu.VMEM((L,), jnp.float32),
            pltpu.SemaphoreType.DMA((2, 2)),
            pltpu.SemaphoreType.DMA((2,)),
        ],
    )


def kernel(c1, c2, edge_index):
    c1f = c1.reshape(B * N, D)
    c2f = c2.reshape(B * N, D)
    idx = jnp.concatenate([edge_index[1].astype(jnp.int32),
                           jnp.zeros((2 * (NPAD - N),), jnp.int32)])

    up, p2 = _sc_u_call()(c1f, c2f, idx)
    sq, sp = _dense_call(c1, c2)
    dot = _dot_call(up, sp)

    df0 = c2[0, 0, :] - c1[0, 0, :]
    pad_corr = 4.0 * (NPAD - N) * jnp.sum(df0 * df0)

    acc2 = jnp.sum(p2) - pad_corr
    return (sq[0, 0] - dot[0, 0] + acc2) / NTOT


# speedup vs baseline: 1.2398x; 1.2398x over previous
import functools

import jax
import jax.numpy as jnp
from jax import lax
from jax.experimental import pallas as pl
from jax.experimental.pallas import tpu as pltpu
from jax.experimental.pallas import tpu_sc as plsc

B = 4
N = 50000
D = 128
NC, NS, L = 2, 16, 16
NW = NC * NS
ROWS_PER_W = 1568
NPAD = NW * ROWS_PER_W
TC_BLK = 1568
K = 56
G = 2 * K
NBUF = 3
NTILES = ROWS_PER_W // K
NTOT = B * N * D


def _bf16_bits(x):
    h = lax.bitcast_convert_type(x.astype(jnp.bfloat16), jnp.uint16)
    return h.astype(jnp.uint32)


def _dense_body(c1_ref, c2_ref, sq_ref, p_ref):
    i = pl.program_id(0)
    d = c2_ref[...] - c1_ref[...]
    row = lax.broadcasted_iota(jnp.int32, (1, TC_BLK, 1), 1) + i * TC_BLK
    d = jnp.where(row < N, d, 0.0)
    packed = (_bf16_bits(d[0]) << 16) | _bf16_bits(jnp.sum(d, axis=0))
    p_ref[...] = lax.bitcast_convert_type(packed, jnp.int32)

    @pl.when(i == 0)
    def _():
        sq_ref[...] = jnp.zeros_like(sq_ref)

    sq_ref[...] += jnp.sum(d * d)


_dense_call = pl.pallas_call(
    _dense_body,
    grid=(NPAD // TC_BLK,),
    in_specs=[
        pl.BlockSpec((B, TC_BLK, D), lambda i: (0, i, 0)),
        pl.BlockSpec((B, TC_BLK, D), lambda i: (0, i, 0)),
    ],
    out_specs=[
        pl.BlockSpec((1, 1), lambda i: (0, 0)),
        pl.BlockSpec((TC_BLK, D), lambda i: (i, 0)),
    ],
    out_shape=[
        jax.ShapeDtypeStruct((1, 1), jnp.float32),
        jax.ShapeDtypeStruct((NPAD, D), jnp.int32),
    ],
)


def _lo_f32(w):
    return lax.bitcast_convert_type(w << 16, jnp.float32)


def _hi_f32(w):
    return lax.bitcast_convert_type(w & jnp.int32(-65536), jnp.float32)


def _sc_gather_body(p_hbm, idx_hbm, out_hbm,
                    idx_v, rg, rs, outv, sems):
    wid = lax.axis_index("s") * NC + lax.axis_index("c")
    base = wid * ROWS_PER_W

    pltpu.sync_copy(idx_hbm.at[pl.ds(2 * base, 2 * ROWS_PER_W)], idx_v)

    def fire(t):
        buf = t % NBUF
        return (
            pltpu.async_copy(p_hbm.at[idx_v.at[pl.ds(t * G, G)]],
                             rg.at[buf], sems.at[buf, 0]),
            pltpu.async_copy(p_hbm.at[pl.ds(base + t * K, K)],
                             rs.at[buf], sems.at[buf, 1]),
        )

    acc1 = jnp.zeros((L,), jnp.float32)
    acc2 = jnp.zeros((L,), jnp.float32)
    handles = {t: fire(t) for t in range(NBUF - 1)}
    for t in range(NTILES):
        if t + NBUF - 1 < NTILES:
            handles[t + NBUF - 1] = fire(t + NBUF - 1)
        for h in handles.pop(t):
            h.wait()
        buf = t % NBUF

        def row_body(r, racc, buf=buf):
            r1, r2 = racc
            for c in range(D // L):
                sl = pl.ds(c * L, L)
                u = _hi_f32(rg[buf, 2 * r, sl]) + _hi_f32(rg[buf, 2 * r + 1, sl])
                r1 = r1 + u * _lo_f32(rs[buf, r, sl])
                r2 = r2 + u * u
            return (r1, r2)

        acc1, acc2 = lax.fori_loop(0, K, row_body, (acc1, acc2))

    outv[0, :] = acc1
    outv[1, :] = acc2
    pltpu.sync_copy(outv, out_hbm.at[wid])


@functools.cache
def _sc_gather_call():
    mesh = plsc.VectorSubcoreMesh(core_axis_name="c", subcore_axis_name="s")
    return pl.kernel(
        _sc_gather_body,
        out_type=jax.ShapeDtypeStruct((NW, 2, L), jnp.float32),
        mesh=mesh,
        scratch_types=[
            pltpu.VMEM((2 * ROWS_PER_W,), jnp.int32),
            pltpu.VMEM((NBUF, G, D), jnp.int32),
            pltpu.VMEM((NBUF, K, D), jnp.int32),
            pltpu.VMEM((2, L), jnp.float32),
            pltpu.SemaphoreType.DMA((NBUF, 2)),
        ],
    )


def kernel(c1, c2, edge_index):
    sq, p = _dense_call(c1, c2)
    idx = jnp.concatenate([edge_index[1].astype(jnp.int32),
                           jnp.full((2 * (NPAD - N),), N, jnp.int32)])
    partials = _sc_gather_call()(p, idx)
    acc1 = jnp.sum(partials[:, 0, :])
    acc2 = jnp.sum(partials[:, 1, :])
    return (sq[0, 0] - acc1 + acc2) / NTOT


# speedup vs baseline: 1.2523x; 1.0101x over previous
import functools

import jax
import jax.numpy as jnp
from jax import lax
from jax.experimental import pallas as pl
from jax.experimental.pallas import tpu as pltpu
from jax.experimental.pallas import tpu_sc as plsc

B = 4
N = 50000
D = 128
NC, NS, L = 2, 16, 16
NW = NC * NS
ROWS_PER_W = 1568
NPAD = NW * ROWS_PER_W
TC_BLK = 1568
K = 56
G = 2 * K
NBUF = 4
NTILES = ROWS_PER_W // K
NTOT = B * N * D


def _bf16_bits(x):
    h = lax.bitcast_convert_type(x.astype(jnp.bfloat16), jnp.uint16)
    return h.astype(jnp.uint32)


def _dense_body(c1_ref, c2_ref, sq_ref, p_ref):
    i = pl.program_id(0)
    d = c2_ref[...] - c1_ref[...]
    row = lax.broadcasted_iota(jnp.int32, (1, TC_BLK, 1), 1) + i * TC_BLK
    d = jnp.where(row < N, d, 0.0)
    packed = (_bf16_bits(d[0]) << 16) | _bf16_bits(jnp.sum(d, axis=0))
    p_ref[...] = lax.bitcast_convert_type(packed, jnp.int32)

    @pl.when(i == 0)
    def _():
        sq_ref[...] = jnp.zeros_like(sq_ref)

    sq_ref[...] += jnp.sum(d * d)


_dense_call = pl.pallas_call(
    _dense_body,
    grid=(NPAD // TC_BLK,),
    in_specs=[
        pl.BlockSpec((B, TC_BLK, D), lambda i: (0, i, 0)),
        pl.BlockSpec((B, TC_BLK, D), lambda i: (0, i, 0)),
    ],
    out_specs=[
        pl.BlockSpec((1, 1), lambda i: (0, 0)),
        pl.BlockSpec((TC_BLK, D), lambda i: (i, 0)),
    ],
    out_shape=[
        jax.ShapeDtypeStruct((1, 1), jnp.float32),
        jax.ShapeDtypeStruct((NPAD, D), jnp.int32),
    ],
)


def _lo_f32(w):
    return lax.bitcast_convert_type(w << 16, jnp.float32)


def _hi_f32(w):
    return lax.bitcast_convert_type(w & jnp.int32(-65536), jnp.float32)


def _sc_gather_body(p_hbm, idx_hbm, out_hbm,
                    idx_v, rg, rs, outv, sems):
    wid = lax.axis_index("s") * NC + lax.axis_index("c")
    base = wid * ROWS_PER_W

    pltpu.sync_copy(idx_hbm.at[pl.ds(2 * base, 2 * ROWS_PER_W)], idx_v)

    def fire(t):
        buf = t % NBUF
        return (
            pltpu.async_copy(p_hbm.at[idx_v.at[pl.ds(t * G, G)]],
                             rg.at[buf], sems.at[buf, 0]),
            pltpu.async_copy(p_hbm.at[pl.ds(base + t * K, K)],
                             rs.at[buf], sems.at[buf, 1]),
        )

    acc1 = jnp.zeros((L,), jnp.float32)
    acc2 = jnp.zeros((L,), jnp.float32)
    handles = {t: fire(t) for t in range(NBUF - 1)}
    for t in range(NTILES):
        if t + NBUF - 1 < NTILES:
            handles[t + NBUF - 1] = fire(t + NBUF - 1)
        for h in handles.pop(t):
            h.wait()
        buf = t % NBUF

        def row_body(r, racc, buf=buf):
            r1, r2 = racc
            for c in range(D // L):
                sl = pl.ds(c * L, L)
                u = _hi_f32(rg[buf, 2 * r, sl]) + _hi_f32(rg[buf, 2 * r + 1, sl])
                r1 = r1 + u * _lo_f32(rs[buf, r, sl])
                r2 = r2 + u * u
            return (r1, r2)

        acc1, acc2 = lax.fori_loop(0, K, row_body, (acc1, acc2))

    outv[0, :] = acc1
    outv[1, :] = acc2
    pltpu.sync_copy(outv, out_hbm.at[wid])


@functools.cache
def _sc_gather_call():
    mesh = plsc.VectorSubcoreMesh(core_axis_name="c", subcore_axis_name="s")
    return pl.kernel(
        _sc_gather_body,
        out_type=jax.ShapeDtypeStruct((NW, 2, L), jnp.float32),
        mesh=mesh,
        scratch_types=[
            pltpu.VMEM((2 * ROWS_PER_W,), jnp.int32),
            pltpu.VMEM((NBUF, G, D), jnp.int32),
            pltpu.VMEM((NBUF, K, D), jnp.int32),
            pltpu.VMEM((2, L), jnp.float32),
            pltpu.SemaphoreType.DMA((NBUF, 2)),
        ],
    )


def kernel(c1, c2, edge_index):
    sq, p = _dense_call(c1, c2)
    idx = jnp.concatenate([edge_index[1].astype(jnp.int32),
                           jnp.full((2 * (NPAD - N),), N, jnp.int32)])
    partials = _sc_gather_call()(p, idx)
    acc1 = jnp.sum(partials[:, 0, :])
    acc2 = jnp.sum(partials[:, 1, :])
    return (sq[0, 0] - acc1 + acc2) / NTOT
